# baseline (speedup 1.0000x reference)
import jax
import jax.numpy as jnp
from jax.experimental import pallas as pl
from jax.experimental.pallas import tpu as pltpu

_N_E = 1024
_E_DIM = 64
_DEPTH = 6
_BM = 1024
_CHUNK = 128


def _vq_block(zft_ref, sw_ref, tw_ref, swt_ref, twt_ref,
              zqs_ref, zqt_ref, inds_ref, indt_ref):
    xt = zft_ref[...]
    rowi = jax.lax.broadcasted_iota(jnp.int32, (_N_E, _BM), 0)
    sw = sw_ref[...]
    tw = tw_ref[...]
    swt = swt_ref[...]
    twt = twt_ref[...]
    sw_sq = jnp.sum(sw ** 2, axis=1, keepdims=True)
    tw_sq = jnp.sum(tw ** 2, axis=1, keepdims=True)
    sw_m2 = -2.0 * sw
    tw_m2 = -2.0 * tw
    res_s = xt
    res_t = xt
    for depth in range(_DEPTH):
        def step(rest, cb_m2, cbt, cb_sq, ind_ref):
            s1 = jnp.sum(rest ** 2, axis=0, keepdims=True)
            dt = ((s1 + cb_sq)
                  + jax.lax.dot_general(
                      cb_m2, rest, (((1,), (0,)), ((), ())),
                      preferred_element_type=jnp.float32))
            dmin = jnp.min(dt, axis=0, keepdims=True)
            mi = jnp.min(jnp.where(dt <= dmin, rowi, _N_E), axis=0)
            hi = mi >> 7
            idx = jnp.broadcast_to((mi & (_CHUNK - 1))[None, :],
                                   (_E_DIM, _BM))
            delta = jnp.zeros((_E_DIM, _BM), jnp.float32)
            for ck in range(_N_E // _CHUNK):
                g = jnp.take_along_axis(
                    cbt[:, ck * _CHUNK:(ck + 1) * _CHUNK], idx, axis=1)
                delta = jnp.where((hi == ck)[None, :], g, delta)
            ind_ref[0, depth, :] = mi
            return rest - delta

        res_s = step(res_s, sw_m2, swt, sw_sq, inds_ref)
        res_t = step(res_t, tw_m2, twt, tw_sq, indt_ref)
    zqs_ref[...] = xt - res_s
    zqt_ref[...] = xt - res_t


def _cos_block(sw_ref, tw_ref, out_ref):
    sw = sw_ref[...]
    tw = tw_ref[...]
    sn = sw / (jnp.sqrt(jnp.sum(sw * sw, axis=1, keepdims=True)) + 1e-8)
    tn = tw / (jnp.sqrt(jnp.sum(tw * tw, axis=1, keepdims=True)) + 1e-8)
    m = jnp.dot(sn, tn.T, preferred_element_type=jnp.float32)
    out_ref[...] = (jnp.sum(m * m) / (_N_E * _N_E))[None, None]


def kernel(z, shared_w, task_w):
    b, c, h, w = z.shape
    ks = 2
    lh, lw = h - ks + 1, w - ks + 1
    feats = [z[:, cc, i:i + lh, j:j + lw]
             for cc in range(c)
             for i in range(ks) for j in range(ks)]
    zft = jnp.stack(feats, axis=0).reshape(_E_DIM, -1)
    n = zft.shape[1]
    nblk = (n + _BM - 1) // _BM
    npad = nblk * _BM
    zft = jnp.pad(zft, ((0, 0), (0, npad - n)))

    zqs_t, zqt_t, inds_blk, indt_blk = pl.pallas_call(
        _vq_block,
        grid=(nblk,),
        compiler_params=pltpu.CompilerParams(
            dimension_semantics=("parallel",)),
        in_specs=[
            pl.BlockSpec((_E_DIM, _BM), lambda i: (0, i)),
            pl.BlockSpec((_N_E, _E_DIM), lambda i: (0, 0)),
            pl.BlockSpec((_N_E, _E_DIM), lambda i: (0, 0)),
            pl.BlockSpec((_E_DIM, _N_E), lambda i: (0, 0)),
            pl.BlockSpec((_E_DIM, _N_E), lambda i: (0, 0)),
        ],
        out_specs=[
            pl.BlockSpec((_E_DIM, _BM), lambda i: (0, i)),
            pl.BlockSpec((_E_DIM, _BM), lambda i: (0, i)),
            pl.BlockSpec((1, _DEPTH, _BM), lambda i: (i, 0, 0)),
            pl.BlockSpec((1, _DEPTH, _BM), lambda i: (i, 0, 0)),
        ],
        out_shape=[
            jax.ShapeDtypeStruct((_E_DIM, npad), jnp.float32),
            jax.ShapeDtypeStruct((_E_DIM, npad), jnp.float32),
            jax.ShapeDtypeStruct((nblk, _DEPTH, _BM), jnp.int32),
            jax.ShapeDtypeStruct((nblk, _DEPTH, _BM), jnp.int32),
        ],
    )(zft, shared_w, task_w, shared_w.T, task_w.T)

    ind_s = (inds_blk.transpose(0, 2, 1).reshape(npad, _DEPTH)[:n]
             .reshape(b, lh, lw, _DEPTH))
    ind_t = (indt_blk.transpose(0, 2, 1).reshape(npad, _DEPTH)[:n]
             .reshape(b, lh, lw, _DEPTH))

    ch = jnp.where((jnp.arange(h) == 0) | (jnp.arange(h) == h - 1), 1.0, 2.0)
    cw = jnp.where((jnp.arange(w) == 0) | (jnp.arange(w) == w - 1), 1.0, 2.0)
    cnt = ch[:, None] * cw[None, :]

    def fold(zq_t_layout):
        zq = (zq_t_layout[:, :n].reshape(_E_DIM, b, lh * lw)
              .transpose(1, 0, 2)
              .reshape(b, c, ks * ks, lh, lw))
        acc = None
        idx = 0
        for i in range(ks):
            for j in range(ks):
                piece = jnp.pad(zq[:, :, idx],
                                ((0, 0), (0, 0),
                                 (i, h - lh - i), (j, w - lw - j)))
                acc = piece if acc is None else acc + piece
                idx += 1
        return acc / cnt

    zq_s_f = fold(zqs_t)
    zq_t_f = fold(zqt_t)
    zq_out = 0.5 * (zq_s_f + zq_t_f)

    cos_loss = pl.pallas_call(
        _cos_block,
        out_shape=jax.ShapeDtypeStruct((1, 1), jnp.float32),
    )(shared_w, task_w)[0, 0]

    beta = 0.25
    loss = ((1.0 + beta) * (jnp.mean((zq_s_f - z) ** 2)
                            + jnp.mean((zq_t_f - z) ** 2))
            + cos_loss)
    return zq_out, loss, ind_s, ind_t


# speedup vs baseline: 1.2838x; 1.2838x over previous
import jax
import jax.numpy as jnp
from jax.experimental import pallas as pl
from jax.experimental.pallas import tpu as pltpu

_N_E = 1024
_E_DIM = 64
_DEPTH = 6
_BM = 512
_CHUNK = 128


def _vq_block(zft_ref, sw_ref, tw_ref, swt_ref, twt_ref,
              zqs_ref, zqt_ref, inds_ref, indt_ref):
    xt = zft_ref[...]
    rowi = jax.lax.broadcasted_iota(jnp.int32, (_N_E, _BM), 0)
    sw = sw_ref[...]
    tw = tw_ref[...]
    swt = swt_ref[...]
    twt = twt_ref[...]
    sw_sq = jnp.sum(sw ** 2, axis=1, keepdims=True)
    tw_sq = jnp.sum(tw ** 2, axis=1, keepdims=True)
    sw_m2 = -2.0 * sw
    tw_m2 = -2.0 * tw
    res_s = xt
    res_t = xt
    for depth in range(_DEPTH):
        def step(rest, cb_m2, cbt, cb_sq, ind_ref):
            s1 = jnp.sum(rest ** 2, axis=0, keepdims=True)
            dt = ((s1 + cb_sq)
                  + jax.lax.dot_general(
                      cb_m2, rest, (((1,), (0,)), ((), ())),
                      preferred_element_type=jnp.float32))
            dmin = jnp.min(dt, axis=0, keepdims=True)
            mi = jnp.min(jnp.where(dt <= dmin, rowi, _N_E), axis=0)
            hi = mi >> 7
            idx = jnp.broadcast_to((mi & (_CHUNK - 1))[None, :],
                                   (_E_DIM, _BM))
            delta = jnp.zeros((_E_DIM, _BM), jnp.float32)
            for ck in range(_N_E // _CHUNK):
                g = jnp.take_along_axis(
                    cbt[:, ck * _CHUNK:(ck + 1) * _CHUNK], idx, axis=1)
                delta = jnp.where((hi == ck)[None, :], g, delta)
            ind_ref[0, depth, :] = mi
            return rest - delta

        res_s = step(res_s, sw_m2, swt, sw_sq, inds_ref)
        res_t = step(res_t, tw_m2, twt, tw_sq, indt_ref)
    zqs_ref[...] = xt - res_s
    zqt_ref[...] = xt - res_t


def _cos_block(sw_ref, tw_ref, out_ref):
    sw = sw_ref[...]
    tw = tw_ref[...]
    sn = sw / (jnp.sqrt(jnp.sum(sw * sw, axis=1, keepdims=True)) + 1e-8)
    tn = tw / (jnp.sqrt(jnp.sum(tw * tw, axis=1, keepdims=True)) + 1e-8)
    m = jnp.dot(sn, tn.T, preferred_element_type=jnp.float32)
    out_ref[...] = (jnp.sum(m * m) / (_N_E * _N_E))[None, None]


def kernel(z, shared_w, task_w):
    b, c, h, w = z.shape
    ks = 2
    lh, lw = h - ks + 1, w - ks + 1
    feats = [z[:, cc, i:i + lh, j:j + lw]
             for cc in range(c)
             for i in range(ks) for j in range(ks)]
    zft = jnp.stack(feats, axis=0).reshape(_E_DIM, -1)
    n = zft.shape[1]
    nblk = (n + _BM - 1) // _BM
    npad = nblk * _BM
    zft = jnp.pad(zft, ((0, 0), (0, npad - n)))

    zqs_t, zqt_t, inds_blk, indt_blk = pl.pallas_call(
        _vq_block,
        grid=(nblk,),
        compiler_params=pltpu.CompilerParams(
            dimension_semantics=("parallel",)),
        in_specs=[
            pl.BlockSpec((_E_DIM, _BM), lambda i: (0, i)),
            pl.BlockSpec((_N_E, _E_DIM), lambda i: (0, 0)),
            pl.BlockSpec((_N_E, _E_DIM), lambda i: (0, 0)),
            pl.BlockSpec((_E_DIM, _N_E), lambda i: (0, 0)),
            pl.BlockSpec((_E_DIM, _N_E), lambda i: (0, 0)),
        ],
        out_specs=[
            pl.BlockSpec((_E_DIM, _BM), lambda i: (0, i)),
            pl.BlockSpec((_E_DIM, _BM), lambda i: (0, i)),
            pl.BlockSpec((1, _DEPTH, _BM), lambda i: (i, 0, 0)),
            pl.BlockSpec((1, _DEPTH, _BM), lambda i: (i, 0, 0)),
        ],
        out_shape=[
            jax.ShapeDtypeStruct((_E_DIM, npad), jnp.float32),
            jax.ShapeDtypeStruct((_E_DIM, npad), jnp.float32),
            jax.ShapeDtypeStruct((nblk, _DEPTH, _BM), jnp.int32),
            jax.ShapeDtypeStruct((nblk, _DEPTH, _BM), jnp.int32),
        ],
    )(zft, shared_w, task_w, shared_w.T, task_w.T)

    ind_s = (inds_blk.transpose(0, 2, 1).reshape(npad, _DEPTH)[:n]
             .reshape(b, lh, lw, _DEPTH))
    ind_t = (indt_blk.transpose(0, 2, 1).reshape(npad, _DEPTH)[:n]
             .reshape(b, lh, lw, _DEPTH))

    ch = jnp.where((jnp.arange(h) == 0) | (jnp.arange(h) == h - 1), 1.0, 2.0)
    cw = jnp.where((jnp.arange(w) == 0) | (jnp.arange(w) == w - 1), 1.0, 2.0)
    cnt = ch[:, None] * cw[None, :]

    def fold(zq_t_layout):
        zq = (zq_t_layout[:, :n].reshape(_E_DIM, b, lh * lw)
              .transpose(1, 0, 2)
              .reshape(b, c, ks * ks, lh, lw))
        acc = None
        idx = 0
        for i in range(ks):
            for j in range(ks):
                piece = jnp.pad(zq[:, :, idx],
                                ((0, 0), (0, 0),
                                 (i, h - lh - i), (j, w - lw - j)))
                acc = piece if acc is None else acc + piece
                idx += 1
        return acc / cnt

    zq_s_f = fold(zqs_t)
    zq_t_f = fold(zqt_t)
    zq_out = 0.5 * (zq_s_f + zq_t_f)

    cos_loss = pl.pallas_call(
        _cos_block,
        out_shape=jax.ShapeDtypeStruct((1, 1), jnp.float32),
    )(shared_w, task_w)[0, 0]

    beta = 0.25
    loss = ((1.0 + beta) * (jnp.mean((zq_s_f - z) ** 2)
                            + jnp.mean((zq_t_f - z) ** 2))
            + cos_loss)
    return zq_out, loss, ind_s, ind_t


# speedup vs baseline: 1.5330x; 1.1941x over previous
import jax
import jax.numpy as jnp
from jax.experimental import pallas as pl
from jax.experimental.pallas import tpu as pltpu

_N_E = 1024
_E_DIM = 64
_DEPTH = 6
_BM = 512
_CHUNK = 128


def _vq_block(zft_ref, sw_ref, tw_ref, sw3t_ref, tw3t_ref,
              zqs_ref, zqt_ref, inds_ref, indt_ref):
    xt = zft_ref[...]
    rowi = jax.lax.broadcasted_iota(jnp.int32, (_N_E, _BM), 0)
    sw = sw_ref[...]
    tw = tw_ref[...]
    sw3t = sw3t_ref[...]
    tw3t = tw3t_ref[...]
    sw_sq = jnp.sum(sw ** 2, axis=1, keepdims=True)
    tw_sq = jnp.sum(tw ** 2, axis=1, keepdims=True)
    sw_m2 = -2.0 * sw
    tw_m2 = -2.0 * tw
    res_s = xt
    res_t = xt
    for depth in range(_DEPTH):
        def step(rest, cb_m2, cb3t, cb_sq, ind_ref):
            s1 = jnp.sum(rest ** 2, axis=0, keepdims=True)
            dt = ((s1 + cb_sq)
                  + jax.lax.dot_general(
                      cb_m2, rest, (((1,), (0,)), ((), ())),
                      preferred_element_type=jnp.float32))
            dmin = jnp.min(dt, axis=0, keepdims=True)
            mi = jnp.min(jnp.where(dt <= dmin, rowi, _N_E), axis=0)
            oh = (rowi == mi[None, :]).astype(jnp.bfloat16)
            g = jax.lax.dot_general(
                cb3t, oh, (((1,), (0,)), ((), ())),
                preferred_element_type=jnp.float32)
            delta = ((g[:_E_DIM] + g[_E_DIM:2 * _E_DIM])
                     + g[2 * _E_DIM:])
            ind_ref[0, depth, :] = mi
            return rest - delta

        res_s = step(res_s, sw_m2, sw3t, sw_sq, inds_ref)
        res_t = step(res_t, tw_m2, tw3t, tw_sq, indt_ref)
    zqs_ref[...] = xt - res_s
    zqt_ref[...] = xt - res_t


def _cos_block(sw_ref, tw_ref, out_ref):
    sw = sw_ref[...]
    tw = tw_ref[...]
    sn = sw / (jnp.sqrt(jnp.sum(sw * sw, axis=1, keepdims=True)) + 1e-8)
    tn = tw / (jnp.sqrt(jnp.sum(tw * tw, axis=1, keepdims=True)) + 1e-8)
    m = jnp.dot(sn, tn.T, preferred_element_type=jnp.float32)
    out_ref[...] = (jnp.sum(m * m) / (_N_E * _N_E))[None, None]


def kernel(z, shared_w, task_w):
    b, c, h, w = z.shape
    ks = 2
    lh, lw = h - ks + 1, w - ks + 1
    feats = [z[:, cc, i:i + lh, j:j + lw]
             for cc in range(c)
             for i in range(ks) for j in range(ks)]
    zft = jnp.stack(feats, axis=0).reshape(_E_DIM, -1)

    def split3t(cb):
        def trunc16(v):
            return jax.lax.bitcast_convert_type(
                jax.lax.bitcast_convert_type(v, jnp.uint32)
                & jnp.uint32(0xFFFF0000), jnp.float32)
        hi = trunc16(cb)
        r1 = cb - hi
        mid = trunc16(r1)
        lo = r1 - mid
        return jnp.concatenate(
            [hi.astype(jnp.bfloat16), mid.astype(jnp.bfloat16),
             lo.astype(jnp.bfloat16)], axis=1).T

    sw3t = split3t(shared_w)
    tw3t = split3t(task_w)
    n = zft.shape[1]
    nblk = (n + _BM - 1) // _BM
    npad = nblk * _BM
    zft = jnp.pad(zft, ((0, 0), (0, npad - n)))

    zqs_t, zqt_t, inds_blk, indt_blk = pl.pallas_call(
        _vq_block,
        grid=(nblk,),
        compiler_params=pltpu.CompilerParams(
            dimension_semantics=("parallel",)),
        in_specs=[
            pl.BlockSpec((_E_DIM, _BM), lambda i: (0, i)),
            pl.BlockSpec((_N_E, _E_DIM), lambda i: (0, 0)),
            pl.BlockSpec((_N_E, _E_DIM), lambda i: (0, 0)),
            pl.BlockSpec((3 * _E_DIM, _N_E), lambda i: (0, 0)),
            pl.BlockSpec((3 * _E_DIM, _N_E), lambda i: (0, 0)),
        ],
        out_specs=[
            pl.BlockSpec((_E_DIM, _BM), lambda i: (0, i)),
            pl.BlockSpec((_E_DIM, _BM), lambda i: (0, i)),
            pl.BlockSpec((1, _DEPTH, _BM), lambda i: (i, 0, 0)),
            pl.BlockSpec((1, _DEPTH, _BM), lambda i: (i, 0, 0)),
        ],
        out_shape=[
            jax.ShapeDtypeStruct((_E_DIM, npad), jnp.float32),
            jax.ShapeDtypeStruct((_E_DIM, npad), jnp.float32),
            jax.ShapeDtypeStruct((nblk, _DEPTH, _BM), jnp.int32),
            jax.ShapeDtypeStruct((nblk, _DEPTH, _BM), jnp.int32),
        ],
    )(zft, shared_w, task_w, sw3t, tw3t)

    ind_s = (inds_blk.transpose(0, 2, 1).reshape(npad, _DEPTH)[:n]
             .reshape(b, lh, lw, _DEPTH))
    ind_t = (indt_blk.transpose(0, 2, 1).reshape(npad, _DEPTH)[:n]
             .reshape(b, lh, lw, _DEPTH))

    ch = jnp.where((jnp.arange(h) == 0) | (jnp.arange(h) == h - 1), 1.0, 2.0)
    cw = jnp.where((jnp.arange(w) == 0) | (jnp.arange(w) == w - 1), 1.0, 2.0)
    cnt = ch[:, None] * cw[None, :]

    def fold(zq_t_layout):
        zq = (zq_t_layout[:, :n].reshape(_E_DIM, b, lh * lw)
              .transpose(1, 0, 2)
              .reshape(b, c, ks * ks, lh, lw))
        acc = None
        idx = 0
        for i in range(ks):
            for j in range(ks):
                piece = jnp.pad(zq[:, :, idx],
                                ((0, 0), (0, 0),
                                 (i, h - lh - i), (j, w - lw - j)))
                acc = piece if acc is None else acc + piece
                idx += 1
        return acc / cnt

    zq_s_f = fold(zqs_t)
    zq_t_f = fold(zqt_t)
    zq_out = 0.5 * (zq_s_f + zq_t_f)

    cos_loss = pl.pallas_call(
        _cos_block,
        out_shape=jax.ShapeDtypeStruct((1, 1), jnp.float32),
    )(shared_w, task_w)[0, 0]

    beta = 0.25
    loss = ((1.0 + beta) * (jnp.mean((zq_s_f - z) ** 2)
                            + jnp.mean((zq_t_f - z) ** 2))
            + cos_loss)
    return zq_out, loss, ind_s, ind_t
